# baseline (speedup 1.0000x reference)
import functools

import jax
import jax.numpy as jnp
from jax import lax
from jax.experimental import pallas as pl

B, D, L = 64, 576, 196
Cc, K = 10, 20
M = Cc * K
BB = 8
CPAD = 128


def _tm_kernel(lit_ref, mask_ref, alpha_ref, vote_ref, out_ref):
    mask = mask_ref[...]
    count = jnp.sum(mask, axis=1)
    mask_b = mask.astype(jnp.bfloat16)
    rows = []
    for b in range(BB):
        lit = lit_ref[b].astype(jnp.bfloat16)
        s = jnp.dot(mask_b, lit, preferred_element_type=jnp.float32)
        rows.append(jnp.max(s, axis=1))
    smax = jnp.stack(rows, axis=0)
    clause = (smax >= count[None, :] - 0.5).astype(jnp.float32)
    weighted = clause * alpha_ref[...]
    out_ref[...] = jnp.dot(weighted, vote_ref[...],
                           preferred_element_type=jnp.float32)


@functools.partial(jax.jit, static_argnames=())
def kernel(literals, clause_mask, alpha):
    mask_f = clause_mask.astype(jnp.float32)
    alpha2 = alpha.reshape(1, M).astype(jnp.float32)
    m_idx = jnp.arange(M)
    cls = m_idx // K
    sign = jnp.where((m_idx % K) < (K // 2), 1.0, -1.0)
    vote = (sign[:, None] *
            (cls[:, None] == jnp.arange(CPAD)[None, :])).astype(jnp.float32)

    out = pl.pallas_call(
        _tm_kernel,
        grid=(B // BB,),
        in_specs=[
            pl.BlockSpec((BB, D, L), lambda i: (i, 0, 0)),
            pl.BlockSpec((M, D), lambda i: (0, 0)),
            pl.BlockSpec((1, M), lambda i: (0, 0)),
            pl.BlockSpec((M, CPAD), lambda i: (0, 0)),
        ],
        out_specs=pl.BlockSpec((BB, CPAD), lambda i: (i, 0)),
        out_shape=jax.ShapeDtypeStruct((B, CPAD), jnp.float32),
    )(literals, mask_f, alpha2, vote)
    return out[:, :Cc]


# speedup vs baseline: 1.0267x; 1.0267x over previous
import functools

import jax
import jax.numpy as jnp
from jax import lax
from jax.experimental import pallas as pl

B, D, L = 64, 576, 196
Cc, K = 10, 20
M = Cc * K
BB = 16
CPAD = 128


def _tm_kernel(lit_ref, mask_ref, alpha_ref, vote_ref, out_ref):
    mask = mask_ref[...]
    count = jnp.sum(mask, axis=1)
    mask_b = mask.astype(jnp.bfloat16)
    rows = []
    for b in range(BB):
        lit = lit_ref[b].astype(jnp.bfloat16)
        s = jnp.dot(mask_b, lit, preferred_element_type=jnp.float32)
        rows.append(jnp.max(s, axis=1))
    smax = jnp.stack(rows, axis=0)
    clause = (smax >= count[None, :] - 0.5).astype(jnp.float32)
    weighted = clause * alpha_ref[...]
    out_ref[...] = jnp.dot(weighted, vote_ref[...],
                           preferred_element_type=jnp.float32)


@functools.partial(jax.jit, static_argnames=())
def kernel(literals, clause_mask, alpha):
    mask_f = clause_mask.astype(jnp.float32)
    alpha2 = alpha.reshape(1, M).astype(jnp.float32)
    m_idx = jnp.arange(M)
    cls = m_idx // K
    sign = jnp.where((m_idx % K) < (K // 2), 1.0, -1.0)
    vote = (sign[:, None] *
            (cls[:, None] == jnp.arange(CPAD)[None, :])).astype(jnp.float32)

    out = pl.pallas_call(
        _tm_kernel,
        grid=(B // BB,),
        in_specs=[
            pl.BlockSpec((BB, D, L), lambda i: (i, 0, 0)),
            pl.BlockSpec((M, D), lambda i: (0, 0)),
            pl.BlockSpec((1, M), lambda i: (0, 0)),
            pl.BlockSpec((M, CPAD), lambda i: (0, 0)),
        ],
        out_specs=pl.BlockSpec((BB, CPAD), lambda i: (i, 0)),
        out_shape=jax.ShapeDtypeStruct((B, CPAD), jnp.float32),
    )(literals, mask_f, alpha2, vote)
    return out[:, :Cc]
